# Initial kernel scaffold; baseline (speedup 1.0000x reference)
import jax
import jax.numpy as jnp
from jax.experimental import pallas as pl


def kernel(input, pos_table, neg_table):
    raise NotImplementedError("write your pallas kernel here")


# speedup vs baseline: 1.9222x; 1.9222x over previous
import functools

import jax
import jax.numpy as jnp
from jax import lax
from jax.experimental import pallas as pl
from jax.experimental.pallas import tpu as pltpu
from jax.experimental.pallas import tpu_sc as plsc

_D = 64
_NW = 32
_CHUNK = 128


@functools.lru_cache(maxsize=None)
def _make_gather(B):
    b_per_w = B // _NW
    n_chunks = b_per_w // _CHUNK
    mesh = plsc.VectorSubcoreMesh(core_axis_name="c", subcore_axis_name="s")

    @functools.partial(
        pl.kernel,
        mesh=mesh,
        compiler_params=pltpu.CompilerParams(use_tc_tiling_on_sc=False),
        out_type=(
            jax.ShapeDtypeStruct((B, _D), jnp.float32),
            jax.ShapeDtypeStruct((B, _D), jnp.float32),
        ),
        scratch_types=[
            pltpu.VMEM((n_chunks, _CHUNK), jnp.int32),
            pltpu.VMEM((_CHUNK, _D), jnp.float32),
            pltpu.VMEM((_CHUNK, _D), jnp.float32),
            pltpu.SemaphoreType.DMA,
        ],
    )
    def gather2(idx_hbm, pos_hbm, neg_hbm, out_p, out_n, idx_v, rows_p, rows_n, sem):
        wid = lax.axis_index("s") * 2 + lax.axis_index("c")
        base = wid * b_per_w
        pltpu.sync_copy(idx_hbm.at[wid], idx_v)

        def step(j, carry):
            pltpu.async_copy(pos_hbm.at[idx_v.at[j]], rows_p, sem).wait()
            pltpu.sync_copy(rows_p, out_p.at[pl.ds(base + j * _CHUNK, _CHUNK)])
            pltpu.async_copy(neg_hbm.at[idx_v.at[j]], rows_n, sem).wait()
            pltpu.sync_copy(rows_n, out_n.at[pl.ds(base + j * _CHUNK, _CHUNK)])
            return carry

        lax.fori_loop(0, n_chunks, step, 0)

    return gather2


def kernel(input, pos_table, neg_table):
    B = input.shape[0] * input.shape[1]
    idx = input.reshape(_NW, B // _NW // _CHUNK, _CHUNK).astype(jnp.int32)
    out_p, out_n = _make_gather(B)(idx, pos_table, neg_table)
    return (out_p.reshape(*input.shape, _D), out_n.reshape(*input.shape, _D))


# speedup vs baseline: 2.1670x; 1.1274x over previous
import functools

import jax
import jax.numpy as jnp
from jax import lax
from jax.experimental import pallas as pl
from jax.experimental.pallas import tpu as pltpu
from jax.experimental.pallas import tpu_sc as plsc

_D = 64
_NW = 32
_CHUNK = 128


_NBUF = 4


@functools.lru_cache(maxsize=None)
def _make_gather(B):
    b_per_w = B // _NW
    n_chunks = b_per_w // _CHUNK
    assert n_chunks % _NBUF == 0
    mesh = plsc.VectorSubcoreMesh(core_axis_name="c", subcore_axis_name="s")

    @functools.partial(
        pl.kernel,
        mesh=mesh,
        compiler_params=pltpu.CompilerParams(use_tc_tiling_on_sc=False),
        out_type=(
            jax.ShapeDtypeStruct((B, _D), jnp.float32),
            jax.ShapeDtypeStruct((B, _D), jnp.float32),
        ),
        scratch_types=[
            pltpu.VMEM((n_chunks, _CHUNK), jnp.int32),
            pltpu.VMEM((_NBUF, _CHUNK, _D), jnp.float32),
            pltpu.VMEM((_NBUF, _CHUNK, _D), jnp.float32),
        ]
        + [pltpu.SemaphoreType.DMA] * (2 * _NBUF),
    )
    def gather2(idx_hbm, pos_hbm, neg_hbm, out_p, out_n, idx_v, rows_p, rows_n, *sems):
        gsem = sems[:_NBUF]
        wsem = sems[_NBUF:]
        wid = lax.axis_index("s") * 2 + lax.axis_index("c")
        base = wid * b_per_w
        pltpu.sync_copy(idx_hbm.at[wid], idx_v)

        def fire(j, b):
            pltpu.async_copy(pos_hbm.at[idx_v.at[j]], rows_p.at[b], gsem[b])
            pltpu.async_copy(neg_hbm.at[idx_v.at[j]], rows_n.at[b], gsem[b])

        def drain_gather(j, b):
            pltpu.make_async_copy(pos_hbm.at[idx_v.at[j]], rows_p.at[b], gsem[b]).wait()
            pltpu.make_async_copy(neg_hbm.at[idx_v.at[j]], rows_n.at[b], gsem[b]).wait()

        for b in range(_NBUF):
            fire(b, b)

        def outer(g0, carry):
            for b in range(_NBUF):
                j = g0 + b
                drain_gather(j, b)
                dst = pl.ds(base + j * _CHUNK, _CHUNK)
                cp_p = pltpu.make_async_copy(rows_p.at[b], out_p.at[dst], wsem[b])
                cp_n = pltpu.make_async_copy(rows_n.at[b], out_n.at[dst], wsem[b])
                cp_p.start()
                cp_n.start()
                cp_p.wait()
                cp_n.wait()

                @pl.when(j + _NBUF < n_chunks)
                def _():
                    fire(j + _NBUF, b)

            return carry

        lax.fori_loop(0, n_chunks // _NBUF, lambda i, c: outer(i * _NBUF, c), 0)

    return gather2


def kernel(input, pos_table, neg_table):
    B = input.shape[0] * input.shape[1]
    idx = input.reshape(_NW, B // _NW // _CHUNK, _CHUNK).astype(jnp.int32)
    out_p, out_n = _make_gather(B)(idx, pos_table, neg_table)
    return (out_p.reshape(*input.shape, _D), out_n.reshape(*input.shape, _D))
